# Initial kernel scaffold; baseline (speedup 1.0000x reference)
import jax
import jax.numpy as jnp
from jax.experimental import pallas as pl


def kernel(x, edge_index, W_in, b_in, Wg0, bg0, Wg1, bg1, Wg2, bg2, W_lat, b_lat, Ws0, bs0, Ws1, bs1, Ws2, bs2):
    raise NotImplementedError("write your pallas kernel here")


# speedup vs baseline: 7.6144x; 7.6144x over previous
import functools

import jax
import jax.numpy as jnp
from jax import lax
from jax.experimental import pallas as pl
from jax.experimental.pallas import tpu as pltpu
from jax.experimental.pallas import tpu_sc as plsc

_N = 10000
_E = 320000
_NC = 2
_NS = 16
_C = 128
_CHUNKS = 79
_EPT = _C * _CHUNKS
_EPAD = _NC * _NS * _EPT
_NPAD = 10112
_ZROWS = _NPAD // _NS

_f32 = jnp.float32


def _sc_mesh():
    return plsc.VectorSubcoreMesh(core_axis_name="c", subcore_axis_name="s")


def _make_edge_pass(D):
    @functools.partial(
        pl.kernel,
        out_type=jax.ShapeDtypeStruct((_NC, _NPAD, D), _f32),
        mesh=_sc_mesh(),
        scratch_types=[
            pltpu.VMEM((_C,), jnp.int32),
            pltpu.VMEM((_C,), jnp.int32),
            pltpu.VMEM((_C, D), _f32),
            pltpu.VMEM_SHARED((_NPAD, D), _f32),
            pltpu.SemaphoreType.DMA,
        ],
    )
    def edge_pass(g_hbm, srcp_hbm, dstp_hbm, zeros_hbm, out_hbm,
                  sidx, didx, rows, acc, sem):
        cid = lax.axis_index("c")
        sid = lax.axis_index("s")
        wid = cid * _NS + sid
        pltpu.sync_copy(zeros_hbm, acc.at[pl.ds(sid * _ZROWS, _ZROWS)])
        plsc.subcore_barrier()
        base = wid * _EPT

        def body(t, carry):
            off = base + t * _C
            pltpu.sync_copy(srcp_hbm.at[pl.ds(off, _C)], sidx)
            pltpu.sync_copy(dstp_hbm.at[pl.ds(off, _C)], didx)
            pltpu.async_copy(g_hbm.at[sidx], rows, sem).wait()
            pltpu.sync_copy(rows, acc.at[didx], add=True)
            return carry

        lax.fori_loop(0, _CHUNKS, body, 0)
        plsc.subcore_barrier()
        pltpu.sync_copy(acc.at[pl.ds(sid * _ZROWS, _ZROWS)],
                        out_hbm.at[cid, pl.ds(sid * _ZROWS, _ZROWS)])

    return edge_pass


_edge_pass_128 = _make_edge_pass(128)


@functools.partial(
    pl.kernel,
    out_type=jax.ShapeDtypeStruct((_NC, _NPAD, 128), _f32),
    mesh=_sc_mesh(),
    scratch_types=[
        pltpu.VMEM((_C,), jnp.int32),
        pltpu.VMEM((_C, 128), _f32),
        pltpu.VMEM_SHARED((_NPAD, 128), _f32),
    ],
)
def _deg_pass(dstp_hbm, ones_hbm, zeros_hbm, out_hbm, didx, ones_v, acc):
    cid = lax.axis_index("c")
    sid = lax.axis_index("s")
    wid = cid * _NS + sid
    pltpu.sync_copy(ones_hbm, ones_v)
    pltpu.sync_copy(zeros_hbm, acc.at[pl.ds(sid * _ZROWS, _ZROWS)])
    plsc.subcore_barrier()
    base = wid * _EPT

    def body(t, carry):
        off = base + t * _C
        pltpu.sync_copy(dstp_hbm.at[pl.ds(off, _C)], didx)
        pltpu.sync_copy(ones_v, acc.at[didx], add=True)
        return carry

    lax.fori_loop(0, _CHUNKS, body, 0)
    plsc.subcore_barrier()
    pltpu.sync_copy(acc.at[pl.ds(sid * _ZROWS, _ZROWS)],
                    out_hbm.at[cid, pl.ds(sid * _ZROWS, _ZROWS)])


_R = 2000


def _dot(a, b):
    return jnp.dot(a, b, preferred_element_type=_f32,
                   precision=lax.Precision.HIGHEST)


def _prep_body(x_ref, d0_ref, d1_ref, wi_ref, bi_ref, w0_ref,
               h0_ref, g0_ref, dv_ref):
    h = _dot(x_ref[...], wi_ref[...]) + bi_ref[...]
    deg = d0_ref[0, :, 0:1] + d1_ref[0, :, 0:1] + 1.0
    dv = lax.rsqrt(deg)
    h0_ref[...] = h
    dv_ref[...] = jnp.broadcast_to(dv, (_R, 128))
    g0_ref[...] = _dot(h, w0_ref[...]) * dv


def _prep(x, degp, w_in, b_in, wg0):
    grid = (_N // _R,)
    row = lambda i: (i, 0)
    full = lambda i: (0, 0)
    return pl.pallas_call(
        _prep_body,
        grid=grid,
        in_specs=[
            pl.BlockSpec((_R, 128), row),
            pl.BlockSpec((1, _R, 128), lambda i: (0, i, 0)),
            pl.BlockSpec((1, _R, 128), lambda i: (1, i, 0)),
            pl.BlockSpec((128, 128), full),
            pl.BlockSpec((1, 128), full),
            pl.BlockSpec((128, 128), full),
        ],
        out_specs=[pl.BlockSpec((_R, 128), row)] * 3,
        out_shape=[jax.ShapeDtypeStruct((_N, 128), _f32)] * 3,
    )(x, degp, degp, w_in, b_in, wg0)


def _make_mid_body(D2):
    def mid_body(s0_ref, s1_ref, g_ref, dv_ref, b_ref, wn_ref,
                 h_ref, gn_ref):
        dv = dv_ref[...]
        pre = (s0_ref[0] + s1_ref[0] + g_ref[...]) * dv + b_ref[...]
        h = jnp.where(pre > 0, pre, 0.2 * pre)
        h_ref[...] = h
        gn_ref[...] = _dot(h, wn_ref[...]) * dv[:, :D2]
    return mid_body


def _mid(sc, g, dv, b, wn, D2):
    grid = (_N // _R,)
    row = lambda i: (i, 0)
    full = lambda i: (0, 0)
    return pl.pallas_call(
        _make_mid_body(D2),
        grid=grid,
        in_specs=[
            pl.BlockSpec((1, _R, 128), lambda i: (0, i, 0)),
            pl.BlockSpec((1, _R, 128), lambda i: (1, i, 0)),
            pl.BlockSpec((_R, 128), row),
            pl.BlockSpec((_R, 128), row),
            pl.BlockSpec((1, 128), full),
            pl.BlockSpec((128, D2), full),
        ],
        out_specs=[pl.BlockSpec((_R, 128), row), pl.BlockSpec((_R, D2), row)],
        out_shape=[jax.ShapeDtypeStruct((_N, 128), _f32),
                   jax.ShapeDtypeStruct((_N, D2), _f32)],
    )(sc, sc, g, dv, b, wn)


def _final_body(s0_ref, s1_ref, g3_ref, dv_ref, bl_ref,
                h0_ref, h1_ref, h2_ref,
                ws0_ref, bs0_ref, ws1_ref, bs1_ref, ws2_ref, bs2_ref,
                out_ref):
    dv = dv_ref[...][:, :64]
    lat = (s0_ref[0][:, :64] + s1_ref[0][:, :64] + g3_ref[:, :64]) * dv
    lat = lat + bl_ref[...]
    lat = lat + _dot(h0_ref[...], ws0_ref[...]) + bs0_ref[...]
    lat = lat + _dot(h1_ref[...], ws1_ref[...]) + bs1_ref[...]
    lat = lat + _dot(h2_ref[...], ws2_ref[...]) + bs2_ref[...]
    out_ref[...] = lat


def _final(sc, g3, dv, b_lat, h0, h1, h2, ws0, bs0, ws1, bs1, ws2, bs2):
    grid = (_N // _R,)
    row = lambda i: (i, 0)
    full = lambda i: (0, 0)
    return pl.pallas_call(
        _final_body,
        grid=grid,
        in_specs=[
            pl.BlockSpec((1, _R, 128), lambda i: (0, i, 0)),
            pl.BlockSpec((1, _R, 128), lambda i: (1, i, 0)),
            pl.BlockSpec((_R, 128), row),
            pl.BlockSpec((_R, 128), row),
            pl.BlockSpec((1, 64), full),
            pl.BlockSpec((_R, 128), row),
            pl.BlockSpec((_R, 128), row),
            pl.BlockSpec((_R, 128), row),
            pl.BlockSpec((128, 64), full),
            pl.BlockSpec((1, 64), full),
            pl.BlockSpec((128, 64), full),
            pl.BlockSpec((1, 64), full),
            pl.BlockSpec((128, 64), full),
            pl.BlockSpec((1, 64), full),
        ],
        out_specs=pl.BlockSpec((_R, 64), row),
        out_shape=jax.ShapeDtypeStruct((_N, 64), _f32),
    )(sc, sc, g3, dv, b_lat, h0, h1, h2, ws0, bs0, ws1, bs1, ws2, bs2)


def kernel(x, edge_index, W_in, b_in, Wg0, bg0, Wg1, bg1, Wg2, bg2,
           W_lat, b_lat, Ws0, bs0, Ws1, bs1, Ws2, bs2):
    src = edge_index[0]
    dst = edge_index[1]
    pad = _EPAD - _E
    srcp = jnp.concatenate([src, jnp.zeros((pad,), src.dtype)])
    dstp = jnp.concatenate([dst, jnp.full((pad,), _N, dst.dtype)])

    zeros128 = jnp.zeros((_ZROWS, 128), _f32)
    ones128 = jnp.ones((_C, 128), _f32)
    wlat128 = jnp.concatenate([W_lat, jnp.zeros((128, 64), _f32)], axis=1)

    degp = _deg_pass(dstp, ones128, zeros128)
    h0, g0, dv = _prep(x, degp, W_in, jnp.reshape(b_in, (1, 128)), Wg0)

    sc = _edge_pass_128(g0, srcp, dstp, zeros128)
    h1, g1 = _mid(sc, g0, dv, jnp.reshape(bg0, (1, 128)), Wg1, 128)

    sc = _edge_pass_128(g1, srcp, dstp, zeros128)
    h2, g2 = _mid(sc, g1, dv, jnp.reshape(bg1, (1, 128)), Wg2, 128)

    sc = _edge_pass_128(g2, srcp, dstp, zeros128)
    h3, g3 = _mid(sc, g2, dv, jnp.reshape(bg2, (1, 128)), wlat128, 128)

    sc = _edge_pass_128(g3, srcp, dstp, zeros128)
    latent = _final(sc, g3, dv, jnp.reshape(b_lat, (1, 64)),
                    h0, h1, h2,
                    Ws0, jnp.reshape(bs0, (1, 64)),
                    Ws1, jnp.reshape(bs1, (1, 64)),
                    Ws2, jnp.reshape(bs2, (1, 64)))
    return latent
